# baseline (speedup 1.0000x reference)
import jax
import jax.numpy as jnp
from jax import lax
from jax.experimental import pallas as pl
from jax.experimental.pallas import tpu as pltpu, tpu_sc as plsc

_B, _C, _H, _W = 4, 9, 512, 512
_D = 16
_GRID_N = _D * _D * _D
_NPIX = _H * _W
_NC, _NS = 2, 16
_NW = _NC * _NS
_PPW = _B * _NPIX // _NW
_CH = 8192
_L = 16
_PROJ_OFF = 0


def _proj_body(w_ref, g_ref, r_ref, o_ref):
    o_ref[...] = lax.dot_general(
        w_ref[...], g_ref[...], (((1,), (0,)), ((), ())),
        precision=lax.Precision.HIGHEST,
        preferred_element_type=jnp.float32) + r_ref[...]


def _sc_body(proj_hbm, guide_hbm, out_hbm,
             p0, p1, p2, gr0, gg0, gb0, gr1, gg1, gb1,
             or0, og0, ob0, or1, og1, ob1, sin0, sin1, sout0, sout1):
    wid = lax.axis_index("s") * _NC + lax.axis_index("c")
    wpb = _NW // _B
    batch = wid // wpb
    slot = wid % wpb

    for o, pref in enumerate((p0, p1, p2)):
        pltpu.sync_copy(
            proj_hbm.at[pl.ds((o * _B + batch) * _GRID_N, _GRID_N)],
            pref)

    gins = ((gr0, gg0, gb0), (gr1, gg1, gb1))
    obufs = ((or0, og0, ob0), (or1, og1, ob1))
    sins, souts = (sin0, sin1), (sout0, sout1)
    nchunks = _PPW // _CH

    def start_in(ci):
        base = slot * _PPW + ci * _CH
        return [pltpu.async_copy(
            guide_hbm.at[pl.ds((batch * 3 + k) * _NPIX + base, _CH)],
            gins[ci % 2][k], sins[ci % 2]) for k in range(3)]

    def start_out(ci):
        base = slot * _PPW + ci * _CH
        return [pltpu.async_copy(
            obufs[ci % 2][k],
            out_hbm.at[pl.ds((batch * 3 + k) * _NPIX + base, _CH)],
            souts[ci % 2]) for k in range(3)]

    h_in = {0: start_in(0)}
    h_out = {}
    for ci in range(nchunks):
        if ci + 1 < nchunks:
            h_in[ci + 1] = start_in(ci + 1)
        for h in h_in.pop(ci):
            h.wait()
        if ci >= 2:
            for h in h_out.pop(ci - 2):
                h.wait()
        gbuf = gins[ci % 2]
        obuf = obufs[ci % 2]

        def vbody(v, carry):
            for s in (v * (2 * _L), v * (2 * _L) + _L):
                rv = gbuf[0][pl.ds(s, _L)]
                gv = gbuf[1][pl.ds(s, _L)]
                bv = gbuf[2][pl.ds(s, _L)]

                def prep(x):
                    xc = x * (_D - 1.0)
                    i0 = xc.astype(jnp.int32)
                    f = xc - i0.astype(jnp.float32)
                    return i0, i0 + 1, f

                r0, r1, fr = prep(rv)
                g0, g1, fg = prep(gv)
                b0, b1, fb = prep(bv)

                rA = r0 * 256
                rB = r1 * 256
                gA = g0 * 16
                gB = g1 * 16
                e00 = rA + gA
                e01 = rA + gB
                e10 = rB + gA
                e11 = rB + gB
                addr = (e00 + b0, e00 + b1, e01 + b0, e01 + b1,
                        e10 + b0, e10 + b1, e11 + b0, e11 + b1)

                frm = 1.0 - fr
                fgm = 1.0 - fg
                fbm = 1.0 - fb
                w00 = frm * fgm
                w01 = frm * fg
                w10 = fr * fgm
                w11 = fr * fg
                wt = (w00 * fbm, w00 * fb, w01 * fbm, w01 * fb,
                      w10 * fbm, w10 * fb, w11 * fbm, w11 * fb)

                for o, pref in enumerate((p0, p1, p2)):
                    t = [plsc.load_gather(pref, [addr[kk]]) * wt[kk]
                         for kk in range(8)]
                    acc = ((t[0] + t[1]) + (t[2] + t[3])) + \
                          ((t[4] + t[5]) + (t[6] + t[7]))
                    obuf[o][pl.ds(s, _L)] = acc
            return carry

        lax.fori_loop(0, _CH // (2 * _L), vbody, None)

        h_out[ci] = start_out(ci)

    for ci in (nchunks - 2, nchunks - 1):
        for h in h_out.pop(ci):
            h.wait()


def kernel(bilateral_grid, guidemap, conv_w, conv_b):
    grid_t = (bilateral_grid.reshape(_B, _C, _GRID_N)
              .transpose(1, 0, 2).reshape(_C, _B * _GRID_N))
    grid_tp = jnp.pad(grid_t, ((0, 16 - _C), (0, 0)))
    w9p = jnp.pad(conv_w[:, :_C], ((0, 5), (0, 16 - _C)))

    cell = jnp.arange(_GRID_N, dtype=jnp.float32)
    coords = jnp.stack([jnp.floor_divide(cell, 256.0) % 16.0,
                        jnp.floor_divide(cell, 16.0) % 16.0,
                        cell % 16.0]) / (_D - 1.0)
    ramp = conv_w[:, _C:] @ coords + conv_b[:, None]
    ramp = jnp.pad(ramp, ((0, 5), (0, 0)))
    ramp = jnp.tile(ramp, (1, _B))

    proj = pl.pallas_call(
        _proj_body,
        out_shape=jax.ShapeDtypeStruct((8, _B * _GRID_N), jnp.float32),
    )(w9p, grid_tp, ramp)
    proj_flat = proj.reshape(-1)

    guide_flat = guidemap.reshape(-1)

    mesh = plsc.VectorSubcoreMesh(core_axis_name="c", subcore_axis_name="s",
                                  num_cores=_NC, num_subcores=_NS)
    scratch = ([pltpu.VMEM((_GRID_N,), jnp.float32) for _ in range(3)]
               + [pltpu.VMEM((_CH,), jnp.float32) for _ in range(12)]
               + [pltpu.SemaphoreType.DMA for _ in range(4)])
    out = pl.kernel(
        _sc_body,
        out_type=jax.ShapeDtypeStruct((_B * 3 * _NPIX,), jnp.float32),
        mesh=mesh,
        scratch_types=scratch,
        compiler_params=pltpu.CompilerParams(needs_layout_passes=False),
    )(proj_flat, guide_flat)
    return out.reshape(_B, 3, _H, _W)


# speedup vs baseline: 1.0543x; 1.0543x over previous
import jax
import jax.numpy as jnp
from jax import lax
from jax.experimental import pallas as pl
from jax.experimental.pallas import tpu as pltpu, tpu_sc as plsc

_B, _C, _H, _W = 4, 9, 512, 512
_D = 16
_GRID_N = _D * _D * _D
_NPIX = _H * _W
_NC, _NS = 2, 16
_NW = _NC * _NS
_PPW = _B * _NPIX // _NW
_CH = 8192
_L = 16
_PROJ_OFF = 0


def _proj_body(w_ref, g_ref, r_ref, o_ref):
    o_ref[...] = lax.dot_general(
        w_ref[...], g_ref[...], (((1,), (0,)), ((), ())),
        precision=lax.Precision.HIGHEST,
        preferred_element_type=jnp.float32) + r_ref[...]


def _sc_body(proj_hbm, guide_hbm, out_hbm,
             p0, p1, p2, gr0, gg0, gb0, gr1, gg1, gb1,
             or0, og0, ob0, or1, og1, ob1, sin0, sin1, sout0, sout1):
    wid = lax.axis_index("s") * _NC + lax.axis_index("c")
    wpb = _NW // _B
    batch = wid // wpb
    slot = wid % wpb

    for o, pref in enumerate((p0, p1, p2)):
        pltpu.sync_copy(
            proj_hbm.at[pl.ds((o * _B + batch) * _GRID_N, _GRID_N)],
            pref)

    gins = ((gr0, gg0, gb0), (gr1, gg1, gb1))
    obufs = ((or0, og0, ob0), (or1, og1, ob1))
    sins, souts = (sin0, sin1), (sout0, sout1)
    nchunks = _PPW // _CH

    def start_in(ci):
        base = slot * _PPW + ci * _CH
        return [pltpu.async_copy(
            guide_hbm.at[pl.ds((batch * 3 + k) * _NPIX + base, _CH)],
            gins[ci % 2][k], sins[ci % 2]) for k in range(3)]

    def start_out(ci):
        base = slot * _PPW + ci * _CH
        return [pltpu.async_copy(
            obufs[ci % 2][k],
            out_hbm.at[pl.ds((batch * 3 + k) * _NPIX + base, _CH)],
            souts[ci % 2]) for k in range(3)]

    h_in = {0: start_in(0)}
    h_out = {}
    for ci in range(nchunks):
        if ci + 1 < nchunks:
            h_in[ci + 1] = start_in(ci + 1)
        for h in h_in.pop(ci):
            h.wait()
        if ci >= 2:
            for h in h_out.pop(ci - 2):
                h.wait()
        gbuf = gins[ci % 2]
        obuf = obufs[ci % 2]

        def vbody(v, carry):
            for s in (v * _L,):
                rv = gbuf[0][pl.ds(s, _L)]
                gv = gbuf[1][pl.ds(s, _L)]
                bv = gbuf[2][pl.ds(s, _L)]

                def prep(x):
                    xc = x * (_D - 1.0)
                    i0 = xc.astype(jnp.int32)
                    f = xc - i0.astype(jnp.float32)
                    return i0, i0 + 1, f

                r0, r1, fr = prep(rv)
                g0, g1, fg = prep(gv)
                b0, b1, fb = prep(bv)

                rA = r0 * 256
                rB = r1 * 256
                gA = g0 * 16
                gB = g1 * 16
                e00 = rA + gA
                e01 = rA + gB
                e10 = rB + gA
                e11 = rB + gB
                addr = (e00 + b0, e00 + b1, e01 + b0, e01 + b1,
                        e10 + b0, e10 + b1, e11 + b0, e11 + b1)

                frm = 1.0 - fr
                fgm = 1.0 - fg
                fbm = 1.0 - fb
                w00 = frm * fgm
                w01 = frm * fg
                w10 = fr * fgm
                w11 = fr * fg
                wt = (w00 * fbm, w00 * fb, w01 * fbm, w01 * fb,
                      w10 * fbm, w10 * fb, w11 * fbm, w11 * fb)

                for o, pref in enumerate((p0, p1, p2)):
                    acc = plsc.load_gather(pref, [addr[0]]) * wt[0]
                    for kk in range(1, 8):
                        acc = acc + plsc.load_gather(pref, [addr[kk]]) * wt[kk]
                    obuf[o][pl.ds(s, _L)] = acc
            return carry

        lax.fori_loop(0, _CH // _L, vbody, None)

        h_out[ci] = start_out(ci)

    for ci in (nchunks - 2, nchunks - 1):
        for h in h_out.pop(ci):
            h.wait()


def kernel(bilateral_grid, guidemap, conv_w, conv_b):
    grid_t = (bilateral_grid.reshape(_B, _C, _GRID_N)
              .transpose(1, 0, 2).reshape(_C, _B * _GRID_N))
    grid_tp = jnp.pad(grid_t, ((0, 16 - _C), (0, 0)))
    w9p = jnp.pad(conv_w[:, :_C], ((0, 5), (0, 16 - _C)))

    cell = jnp.arange(_GRID_N, dtype=jnp.float32)
    coords = jnp.stack([jnp.floor_divide(cell, 256.0) % 16.0,
                        jnp.floor_divide(cell, 16.0) % 16.0,
                        cell % 16.0]) / (_D - 1.0)
    ramp = conv_w[:, _C:] @ coords + conv_b[:, None]
    ramp = jnp.pad(ramp, ((0, 5), (0, 0)))
    ramp = jnp.tile(ramp, (1, _B))

    proj = pl.pallas_call(
        _proj_body,
        out_shape=jax.ShapeDtypeStruct((8, _B * _GRID_N), jnp.float32),
    )(w9p, grid_tp, ramp)
    proj_flat = proj.reshape(-1)

    guide_flat = guidemap.reshape(-1)

    mesh = plsc.VectorSubcoreMesh(core_axis_name="c", subcore_axis_name="s",
                                  num_cores=_NC, num_subcores=_NS)
    scratch = ([pltpu.VMEM((_GRID_N,), jnp.float32) for _ in range(3)]
               + [pltpu.VMEM((_CH,), jnp.float32) for _ in range(12)]
               + [pltpu.SemaphoreType.DMA for _ in range(4)])
    out = pl.kernel(
        _sc_body,
        out_type=jax.ShapeDtypeStruct((_B * 3 * _NPIX,), jnp.float32),
        mesh=mesh,
        scratch_types=scratch,
        compiler_params=pltpu.CompilerParams(needs_layout_passes=False),
    )(proj_flat, guide_flat)
    return out.reshape(_B, 3, _H, _W)
